# Initial kernel scaffold; baseline (speedup 1.0000x reference)
import jax
import jax.numpy as jnp
from jax.experimental import pallas as pl


def kernel(x, adj, W1, b1, W2, b2):
    raise NotImplementedError("write your pallas kernel here")


# speedup vs baseline: 1.0075x; 1.0075x over previous
import functools

import jax
import jax.numpy as jnp
from jax.experimental import pallas as pl
from jax.experimental.pallas import tpu as pltpu


def _pick_bm(n: int) -> int:
    for bm in (400, 200, 100, 40, 8):
        if n % bm == 0 and bm % 8 == 0:
            return bm
    return n


def _gcn_kernel(nb, x_ref, adj_ref, w1_ref, b1_ref, w2_ref, b2_ref,
                ls_ref, sm_ref, s1_ref, s2_ref):
    t = pl.program_id(0)
    bm = adj_ref.shape[0]

    @pl.when(t == 0)
    def _():
        s1_ref[...] = jnp.dot(x_ref[...], w1_ref[...],
                              preferred_element_type=jnp.float32)

    @pl.when(t < nb)
    def _():
        h = jnp.dot(adj_ref[...], s1_ref[...],
                    preferred_element_type=jnp.float32) + b1_ref[...]
        h = jnp.maximum(h, 0.0)
        s2_ref[pl.ds(t * bm, bm), :] = jnp.dot(
            h, w2_ref[...], preferred_element_type=jnp.float32)

    @pl.when(t >= nb)
    def _():
        logits = jnp.dot(adj_ref[...], s2_ref[...],
                         preferred_element_type=jnp.float32) + b2_ref[...]
        m = jnp.max(logits, axis=1, keepdims=True)
        z = logits - m
        e = jnp.exp(z)
        s = jnp.sum(e, axis=1, keepdims=True)
        sm_ref[...] = e / s
        ls_ref[...] = z - jnp.log(s)


def kernel(x, adj, W1, b1, W2, b2):
    n, f_in = x.shape
    h_dim = W1.shape[1]
    c_dim = W2.shape[1]
    bm = _pick_bm(n)
    nb = n // bm

    b1r = b1.reshape(1, h_dim)
    b2r = b2.reshape(1, c_dim)

    out_idx = lambda t: (jnp.where(t < nb, 0, t - nb), 0)

    ls, sm = pl.pallas_call(
        functools.partial(_gcn_kernel, nb),
        grid=(2 * nb,),
        in_specs=[
            pl.BlockSpec((n, f_in), lambda t: (0, 0)),
            pl.BlockSpec((bm, n), lambda t: (t % nb, 0)),
            pl.BlockSpec((f_in, h_dim), lambda t: (0, 0)),
            pl.BlockSpec((1, h_dim), lambda t: (0, 0)),
            pl.BlockSpec((h_dim, c_dim), lambda t: (0, 0)),
            pl.BlockSpec((1, c_dim), lambda t: (0, 0)),
        ],
        out_specs=[
            pl.BlockSpec((bm, c_dim), out_idx),
            pl.BlockSpec((bm, c_dim), out_idx),
        ],
        out_shape=[
            jax.ShapeDtypeStruct((n, c_dim), jnp.float32),
            jax.ShapeDtypeStruct((n, c_dim), jnp.float32),
        ],
        scratch_shapes=[
            pltpu.VMEM((n, h_dim), jnp.float32),
            pltpu.VMEM((n, c_dim), jnp.float32),
        ],
    )(x, adj, W1, b1r, W2, b2r)
    return ls, sm


# speedup vs baseline: 1.0130x; 1.0054x over previous
import functools

import jax
import jax.numpy as jnp
from jax.experimental import pallas as pl
from jax.experimental.pallas import tpu as pltpu


def _pick_bm(n: int) -> int:
    for bm in (400, 200, 100, 40, 8):
        if n % bm == 0 and bm % 8 == 0:
            return bm
    return n


def _gcn_kernel(nb, x_ref, adj_ref, w1_ref, b1_ref, w2_ref, b2_ref,
                ls_ref, sm_ref, s1_ref, s2_ref):
    t = pl.program_id(0)
    bm = adj_ref.shape[0]

    @pl.when(t == 0)
    def _():
        s1_ref[...] = jnp.dot(x_ref[...], w1_ref[...],
                              preferred_element_type=jnp.float32
                              ).astype(jnp.bfloat16)

    @pl.when(t < nb)
    def _():
        h = jnp.dot(adj_ref[...].astype(jnp.bfloat16), s1_ref[...],
                    preferred_element_type=jnp.float32) + b1_ref[...]
        h = jnp.maximum(h, 0.0)
        s2_ref[pl.ds(t * bm, bm), :] = jnp.dot(
            h, w2_ref[...], preferred_element_type=jnp.float32
        ).astype(jnp.bfloat16)

    @pl.when(t >= nb)
    def _():
        logits = jnp.dot(adj_ref[...].astype(jnp.bfloat16), s2_ref[...],
                         preferred_element_type=jnp.float32) + b2_ref[...]
        m = jnp.max(logits, axis=1, keepdims=True)
        z = logits - m
        e = jnp.exp(z)
        s = jnp.sum(e, axis=1, keepdims=True)
        sm_ref[...] = e / s
        ls_ref[...] = z - jnp.log(s)


def kernel(x, adj, W1, b1, W2, b2):
    n, f_in = x.shape
    h_dim = W1.shape[1]
    c_dim = W2.shape[1]
    bm = _pick_bm(n)
    nb = n // bm

    b1r = b1.reshape(1, h_dim)
    b2r = b2.reshape(1, c_dim)

    out_idx = lambda t: (jnp.where(t < nb, 0, t - nb), 0)

    ls, sm = pl.pallas_call(
        functools.partial(_gcn_kernel, nb),
        grid=(2 * nb,),
        in_specs=[
            pl.BlockSpec((n, f_in), lambda t: (0, 0)),
            pl.BlockSpec((bm, n), lambda t: (t % nb, 0)),
            pl.BlockSpec((f_in, h_dim), lambda t: (0, 0)),
            pl.BlockSpec((1, h_dim), lambda t: (0, 0)),
            pl.BlockSpec((h_dim, c_dim), lambda t: (0, 0)),
            pl.BlockSpec((1, c_dim), lambda t: (0, 0)),
        ],
        out_specs=[
            pl.BlockSpec((bm, c_dim), out_idx),
            pl.BlockSpec((bm, c_dim), out_idx),
        ],
        out_shape=[
            jax.ShapeDtypeStruct((n, c_dim), jnp.float32),
            jax.ShapeDtypeStruct((n, c_dim), jnp.float32),
        ],
        scratch_shapes=[
            pltpu.VMEM((n, h_dim), jnp.bfloat16),
            pltpu.VMEM((n, c_dim), jnp.bfloat16),
        ],
    )(x, adj, W1, b1r, W2, b2r)
    return ls, sm
